# baseline (speedup 1.0000x reference)
import functools
import math

import jax
import jax.numpy as jnp
from jax import lax
from jax.experimental import pallas as pl
from jax.experimental.pallas import tpu as pltpu
from jax.experimental.pallas import tpu_sc as plsc

NROW, NCOL = 4096, 200
V = 1000000
D = 64
LANES = 16
NC, NS = 2, 16
NW = NC * NS
SCALE = math.sqrt(D)
VP = V // 2

CB = 384
NBLK = V // CB
TAIL0 = V - 128
PTAIL = TAIL0 // 2
KMAX = -(-NBLK // NW)

G = NCOL

_mesh = plsc.VectorSubcoreMesh(core_axis_name="c", subcore_axis_name="s")


def _transpose_pairs(src, dst, b, nrows):
    lanes = lax.iota(jnp.int32, LANES)

    @plsc.parallel_loop(0, nrows, 1, unroll=2)
    def _(p):
        for h in (0, 1):
            col = jnp.full((LANES,), 2 * p + h, dtype=jnp.int32)
            for k in range(D // LANES):
                v = plsc.load_gather(src.at[b], [lanes + k * LANES, col])
                dst[b, p, pl.ds(h * D + k * LANES, LANES)] = v


@functools.partial(
    pl.kernel,
    mesh=_mesh,
    out_type=jax.ShapeDtypeStruct((VP, 128), jnp.float32),
    scratch_types=[
        pltpu.VMEM((2, D, CB), jnp.float32),
        pltpu.VMEM((2, CB // 2, 128), jnp.float32),
        pltpu.SemaphoreType.DMA,
        pltpu.SemaphoreType.DMA,
        pltpu.SemaphoreType.DMA,
        pltpu.SemaphoreType.DMA,
    ],
    compiler_params=pltpu.CompilerParams(use_tc_tiling_on_sc=True, needs_layout_passes=False),
)
def _pack(tab_hbm, tail_hbm, packed_hbm, bufr, bufw, r0, r1, w0, w1):
    wid = lax.axis_index("s") * NC + lax.axis_index("c")
    rsem = (r0, r1)
    wsem = (w0, w1)

    def blk_off(k):
        return pl.multiple_of((wid + k * NW) * CB, 128)

    def read(k, b):
        pltpu.async_copy(
            tab_hbm.at[:, pl.ds(blk_off(k), CB)], bufr.at[b], rsem[b]
        )

    def wait_read(b):
        pltpu.make_async_copy(
            tab_hbm.at[:, pl.ds(0, CB)], bufr.at[b], rsem[b]
        ).wait()

    def write(k, b):
        p0 = pl.multiple_of((wid + k * NW) * (CB // 2), 8)
        pltpu.async_copy(
            bufw.at[b], packed_hbm.at[pl.ds(p0, CB // 2)], wsem[b]
        )

    def wait_write(b):
        pltpu.make_async_copy(
            bufw.at[b], packed_hbm.at[pl.ds(0, CB // 2)], wsem[b]
        ).wait()

    nk = jnp.where(wid < NBLK - (KMAX - 1) * NW, KMAX, KMAX - 1)
    read(0, 0)

    def body(k, carry):
        for b in (0, 1):
            kc = 2 * k + b

            @pl.when(kc < nk)
            def _():
                @pl.when(kc + 1 < nk)
                def _():
                    read(kc + 1, 1 - b)

                wait_read(b)

                @pl.when(kc >= 2)
                def _():
                    wait_write(b)

                _transpose_pairs(bufr, bufw, b, CB // 2)
                write(kc, b)
        return carry

    lax.fori_loop(0, (KMAX + 1) // 2, body, 0)

    @pl.when(nk >= 2)
    def _():
        wait_write(0)
        wait_write(1)

    @pl.when(nk == 1)
    def _():
        wait_write(0)

    @pl.when(wid == 0)
    def _():
        pltpu.sync_copy(tail_hbm, bufr.at[0, :, pl.ds(0, 128)])
        _transpose_pairs(bufr, bufw, 0, D)
        pltpu.sync_copy(
            bufw.at[0, pl.ds(0, D)],
            packed_hbm.at[pl.ds(PTAIL, D)],
        )


@functools.partial(
    pl.kernel,
    mesh=_mesh,
    out_type=jax.ShapeDtypeStruct((NCOL, D, NROW), jnp.float32),
    scratch_types=[
        pltpu.VMEM((2, 8, 128), jnp.int32),
        pltpu.VMEM((2, 1, 128), jnp.int32),
        pltpu.VMEM((2, 1, 128), jnp.int32),
        pltpu.VMEM((2, 128, 128), jnp.float32),
        pltpu.VMEM((2, 1, D, 128), jnp.float32),
        pltpu.SemaphoreType.DMA,
        pltpu.SemaphoreType.DMA,
        pltpu.SemaphoreType.DMA,
        pltpu.SemaphoreType.DMA,
    ],
    compiler_params=pltpu.CompilerParams(use_tc_tiling_on_sc=True, needs_layout_passes=False),
)
def _lookup(x_hbm, packed_hbm, out_hbm, idxr, j2, par, rows, outb,
            g0, g1, o0, o1):
    wid = lax.axis_index("s") * NC + lax.axis_index("c")
    ib0 = pl.multiple_of(wid * 128, 128)
    gsem = (g0, g1)
    osem = (o0, o1)

    def load_group(j):
        grp = j // 8
        gb = grp % 2
        pltpu.sync_copy(
            x_hbm.at[pl.ds(pl.multiple_of(grp * 8, 8), 8), pl.ds(ib0, 128)],
            idxr.at[gb],
        )

    def prep(j, b):
        gb = (j // 8) % 2
        r = j % 8
        for k in range(128 // LANES):
            sl = pl.ds(k * LANES, LANES)
            v = idxr[gb, r, sl]
            j2[b, 0, sl] = v >> 1
            par[b, 0, sl] = (v & 1) << 6

    def fire(j, b):
        pltpu.async_copy(
            packed_hbm.at[j2.at[b, 0]], rows.at[b], gsem[b]
        )

    def wait_gather(b):
        pltpu.make_async_copy(
            packed_hbm.at[j2.at[0, 0]], rows.at[b], gsem[b]
        ).wait()

    def wait_writeback(b):
        pltpu.make_async_copy(
            outb.at[b], out_hbm.at[pl.ds(0, 1), :, pl.ds(0, 128)], osem[b]
        ).wait()

    load_group(0)
    prep(0, 0)
    fire(0, 0)

    lanes = lax.iota(jnp.int32, LANES)

    def pair_body(kk, carry):
        for b in (0, 1):
            j = 2 * kk + b
            nxt = j + 1

            @pl.when(nxt < G)
            def _():
                @pl.when(nxt % 8 == 0)
                def _():
                    load_group(nxt)

                @pl.when(nxt >= 2)
                def _():
                    wait_writeback(1 - b)

                prep(nxt, 1 - b)
                fire(nxt, 1 - b)

            wait_gather(b)

            @plsc.parallel_loop(0, D, 1, unroll=2)
            def _(d):
                for k in range(128 // LANES):
                    sl = pl.ds(k * LANES, LANES)
                    cols = par[b, 0, sl] + d
                    v = plsc.load_gather(
                        rows.at[b], [lanes + k * LANES, cols]
                    )
                    outb[b, 0, d, sl] = v * SCALE

            pltpu.async_copy(
                outb.at[b],
                out_hbm.at[pl.ds(j, 1), :, pl.ds(ib0, 128)],
                osem[b],
            )
        return carry

    lax.fori_loop(0, G // 2, pair_body, 0)
    wait_writeback(0)
    wait_writeback(1)


def kernel(x, table):
    x_t = jnp.transpose(x.astype(jnp.int32))
    tab_t = jnp.transpose(table)
    tail = lax.slice(tab_t, (0, V - 128), (D, V))
    packed = _pack(tab_t, tail)
    out_t = _lookup(x_t, packed)
    return jnp.transpose(out_t, (2, 0, 1))


# speedup vs baseline: 1.2903x; 1.2903x over previous
import functools
import math

import jax
import jax.numpy as jnp
from jax import lax
from jax.experimental import pallas as pl
from jax.experimental.pallas import tpu as pltpu
from jax.experimental.pallas import tpu_sc as plsc

NROW, NCOL = 4096, 200
B = NROW * NCOL
D = 64
LANES = 16
NC, NS = 2, 16
NW = NC * NS
XR = B // 128
RPW = XR // NW
NR = 4
CH = NR * 128
G = RPW // NR
SCALE = math.sqrt(D)

_mesh = plsc.VectorSubcoreMesh(core_axis_name="c", subcore_axis_name="s")


@functools.partial(
    pl.kernel,
    mesh=_mesh,
    out_type=jax.ShapeDtypeStruct((XR, 128), jnp.int32),
    scratch_types=[
        pltpu.VMEM((NCOL, 128), jnp.int32),
        pltpu.VMEM((RPW, 128), jnp.int32),
    ],
    compiler_params=pltpu.CompilerParams(use_tc_tiling_on_sc=True,
                                         needs_layout_passes=False),
)
def _xprep(xt_hbm, out_hbm, b1, b2):
    wid = lax.axis_index("s") * NC + lax.axis_index("c")
    i0 = pl.multiple_of(wid * 128, 128)
    pltpu.sync_copy(xt_hbm.at[:, pl.ds(i0, 128)], b1)
    lanes = lax.iota(jnp.int32, LANES)

    @plsc.parallel_loop(0, RPW, 1, unroll=4)
    def _(rr):
        for k in range(128 // LANES):
            t = rr * 128 + k * LANES + lanes
            v = plsc.load_gather(b1, [t % NCOL, t // NCOL])
            b2[rr, pl.ds(k * LANES, LANES)] = v

    pltpu.sync_copy(b2, out_hbm.at[pl.ds(wid * RPW, RPW)])


@functools.partial(
    pl.kernel,
    mesh=_mesh,
    out_type=jax.ShapeDtypeStruct((B, D), jnp.float32),
    scratch_types=[
        pltpu.VMEM((2, NR, 128), jnp.int32),
        pltpu.VMEM((2, CH, D), jnp.float32),
        pltpu.SemaphoreType.DMA,
        pltpu.SemaphoreType.DMA,
        pltpu.SemaphoreType.DMA,
        pltpu.SemaphoreType.DMA,
    ],
    compiler_params=pltpu.CompilerParams(use_tc_tiling_on_sc=False),
)
def _emb_lookup(x_hbm, table_hbm, out_hbm, idx_v, rows_v, g0, g1, o0, o1):
    wid = lax.axis_index("s") * NC + lax.axis_index("c")
    row_base = wid * RPW
    gsem = (g0, g1)
    osem = (o0, o1)

    def stage(g, b):
        r0 = row_base + g * NR
        pltpu.sync_copy(x_hbm.at[pl.ds(r0, NR)], idx_v.at[b])
        for r in range(NR):
            pltpu.async_copy(
                table_hbm.at[idx_v.at[b, r]],
                rows_v.at[b, pl.ds(r * 128, 128)],
                gsem[b],
            )

    def wait_gathers(b):
        for r in range(NR):
            pltpu.make_async_copy(
                table_hbm.at[idx_v.at[b, r]],
                rows_v.at[b, pl.ds(r * 128, 128)],
                gsem[b],
            ).wait()

    def wait_writeback(b):
        pltpu.make_async_copy(
            rows_v.at[b], out_hbm.at[pl.ds(0, CH)], osem[b]
        ).wait()

    stage(0, 0)

    def pair_body(k, carry):
        for b in (0, 1):
            gc = 2 * k + b
            nxt = gc + 1

            @pl.when(nxt < G)
            def _():
                @pl.when(nxt >= 2)
                def _():
                    wait_writeback(1 - b)

                stage(nxt, 1 - b)

            wait_gathers(b)

            @plsc.parallel_loop(0, CH, 1, unroll=8)
            def _(c):
                for kk in range(D // LANES):
                    sl = pl.ds(kk * LANES, LANES)
                    rows_v[b, c, sl] = rows_v[b, c, sl] * SCALE

            pltpu.async_copy(
                rows_v.at[b],
                out_hbm.at[pl.ds((row_base + gc * NR) * 128, CH)],
                osem[b],
            )
        return carry

    lax.fori_loop(0, G // 2, pair_body, 0)
    wait_writeback(0)
    wait_writeback(1)


def kernel(x, table):
    x_t = jnp.transpose(x.astype(jnp.int32))
    xlin = _xprep(x_t)
    out = _emb_lookup(xlin, table)
    return out.reshape(NROW, NCOL, D)


# speedup vs baseline: 1.3001x; 1.0075x over previous
import functools
import math

import jax
import jax.numpy as jnp
from jax import lax
from jax.experimental import pallas as pl
from jax.experimental.pallas import tpu as pltpu
from jax.experimental.pallas import tpu_sc as plsc

NROW, NCOL = 4096, 200
B = NROW * NCOL
D = 64
LANES = 16
NC, NS = 2, 16
NW = NC * NS
RPW = NROW // NW
NR = 4
CH = NR * NCOL
G = RPW // NR
SCALE = math.sqrt(D)

_mesh = plsc.VectorSubcoreMesh(core_axis_name="c", subcore_axis_name="s")


@functools.partial(
    pl.kernel,
    mesh=_mesh,
    out_type=jax.ShapeDtypeStruct((B, D), jnp.float32),
    scratch_types=[
        pltpu.VMEM((2, NR, NCOL), jnp.int32),
        pltpu.VMEM((2, CH, D), jnp.float32),
        pltpu.SemaphoreType.DMA,
        pltpu.SemaphoreType.DMA,
        pltpu.SemaphoreType.DMA,
        pltpu.SemaphoreType.DMA,
    ],
    compiler_params=pltpu.CompilerParams(use_tc_tiling_on_sc=False),
)
def _emb_lookup(x_hbm, table_hbm, out_hbm, idx_v, rows_v, g0, g1, o0, o1):
    wid = lax.axis_index("s") * NC + lax.axis_index("c")
    row_base = wid * RPW
    gsem = (g0, g1)
    osem = (o0, o1)

    def stage(g, b):
        pltpu.sync_copy(x_hbm.at[pl.ds(row_base + g * NR, NR)], idx_v.at[b])
        for r in range(NR):
            pltpu.async_copy(
                table_hbm.at[idx_v.at[b, r, pl.ds(0, 128)]],
                rows_v.at[b, pl.ds(r * NCOL, 128)],
                gsem[b],
            )
            pltpu.async_copy(
                table_hbm.at[idx_v.at[b, r, pl.ds(128, NCOL - 128)]],
                rows_v.at[b, pl.ds(r * NCOL + 128, NCOL - 128)],
                gsem[b],
            )

    def wait_gathers(b):
        for r in range(NR):
            pltpu.make_async_copy(
                table_hbm.at[idx_v.at[b, r, pl.ds(0, 128)]],
                rows_v.at[b, pl.ds(r * NCOL, 128)],
                gsem[b],
            ).wait()
            pltpu.make_async_copy(
                table_hbm.at[idx_v.at[b, r, pl.ds(128, NCOL - 128)]],
                rows_v.at[b, pl.ds(r * NCOL + 128, NCOL - 128)],
                gsem[b],
            ).wait()

    def wait_writeback(b):
        pltpu.make_async_copy(
            rows_v.at[b], out_hbm.at[pl.ds(0, CH)], osem[b]
        ).wait()

    stage(0, 0)

    def pair_body(k, carry):
        for b in (0, 1):
            gc = 2 * k + b
            nxt = gc + 1

            @pl.when(nxt < G)
            def _():
                @pl.when(nxt >= 2)
                def _():
                    wait_writeback(1 - b)

                stage(nxt, 1 - b)

            wait_gathers(b)

            @plsc.parallel_loop(0, CH, 1, unroll=8)
            def _(r):
                for kk in range(D // LANES):
                    sl = pl.ds(kk * LANES, LANES)
                    rows_v[b, r, sl] = rows_v[b, r, sl] * SCALE

            pltpu.async_copy(
                rows_v.at[b],
                out_hbm.at[pl.ds((row_base + gc * NR) * NCOL, CH)],
                osem[b],
            )
        return carry

    lax.fori_loop(0, G // 2, pair_body, 0)
    wait_writeback(0)
    wait_writeback(1)


def kernel(x, table):
    out = _emb_lookup(x.astype(jnp.int32), table)
    return out.reshape(NROW, NCOL, D)
